# baseline (speedup 1.0000x reference)
import functools

import jax
import jax.numpy as jnp
from jax import lax
from jax.experimental import pallas as pl
from jax.experimental.pallas import tpu as pltpu
from jax.experimental.pallas import tpu_sc as plsc

B = 16384
S = 5
D = 64
H = 128

NC = 2
NS = 16
NW = NC * NS
BPW = B // NW
NIDX = BPW * S

CB = 16
ROWS = CB * S
NCH = BPW // CB


def _sc_pool_body(x_hbm, emb_hbm, out_hbm, idx_v, half_v, coloff_v,
                  rows_v, pooled_v, sem):
    wid = lax.axis_index("s") * NC + lax.axis_index("c")
    base = wid * BPW
    pltpu.sync_copy(x_hbm.at[pl.ds(base * S, NIDX)], idx_v)

    sh = PB.bit_length() - 1

    def prep(k, _):
        sl = pl.ds(k * 16, 16)
        v = idx_v[sl]
        half_v[sl] = lax.bitwise_or(
            lax.shift_left(lax.shift_right_logical(v, sh), sh - 1),
            lax.bitwise_and(v, PB // 2 - 1),
        )
        coloff_v[sl] = lax.shift_left(
            lax.bitwise_and(lax.shift_right_logical(v, sh - 1), 1), 6
        )
        return 0

    lax.fori_loop(0, NIDX // 16, prep, 0)

    def chunk(g, _):
        pltpu.async_copy(
            emb_hbm.at[half_v.at[pl.ds(g * ROWS, ROWS)]], rows_v, sem
        ).wait()

        def pool_one(i, _):
            r = i * S
            cv = coloff_v[pl.ds(g * ROWS + r, 16)]
            c = [cv[s] for s in range(S)]
            for q in range(D // 16):
                acc = rows_v[r, pl.ds(c[0] + q * 16, 16)]
                for s in range(1, S):
                    acc = acc + rows_v[r + s, pl.ds(c[s] + q * 16, 16)]
                pooled_v[g * CB + i, pl.ds(q * 16, 16)] = acc * (1.0 / S)
            return 0

        lax.fori_loop(0, CB, pool_one, 0)
        return 0

    lax.fori_loop(0, NCH, chunk, 0)
    pltpu.sync_copy(pooled_v, out_hbm.at[pl.ds(base, BPW)])


@functools.partial(jax.jit, static_argnames=())
def _sc_pool(x_flat, emb_pairs):
    mesh = plsc.VectorSubcoreMesh(core_axis_name="c", subcore_axis_name="s")
    return pl.kernel(
        _sc_pool_body,
        out_type=jax.ShapeDtypeStruct((B, D), jnp.float32),
        mesh=mesh,
        scratch_types=[
            pltpu.VMEM((NIDX,), jnp.int32),
            pltpu.VMEM((NIDX,), jnp.int32),
            pltpu.VMEM((NIDX + 16,), jnp.int32),
            pltpu.VMEM((ROWS, 2 * D), jnp.float32),
            pltpu.VMEM((BPW, D), jnp.float32),
            pltpu.SemaphoreType.DMA,
        ],
        compiler_params=pltpu.CompilerParams(use_tc_tiling_on_sc=True),
    )(x_flat, emb_pairs)


NP = 1000000
PB = 8192
NBLK = -(-NP // PB)
PACKED_ROWS = NBLK * (PB // 2)


def _pack_body(t_ref, o_ref):
    t = t_ref[...]
    eye = jax.lax.broadcasted_iota(jnp.int32, (D, D), 0) == \
        jax.lax.broadcasted_iota(jnp.int32, (D, D), 1)
    eye = eye.astype(jnp.float32)
    o_ref[:, :D] = jax.lax.dot_general(
        t[:, : PB // 2], eye, (((0,), (0,)), ((), ())),
        precision=jax.lax.Precision.HIGHEST,
        preferred_element_type=jnp.float32)
    o_ref[:, D:] = jax.lax.dot_general(
        t[:, PB // 2 :], eye, (((0,), (0,)), ((), ())),
        precision=jax.lax.Precision.HIGHEST,
        preferred_element_type=jnp.float32)


def _pack(embT):
    return pl.pallas_call(
        _pack_body,
        grid=(NBLK,),
        in_specs=[pl.BlockSpec((D, PB), lambda i: (0, i))],
        out_specs=pl.BlockSpec((PB // 2, 2 * D), lambda i: (i, 0)),
        out_shape=jax.ShapeDtypeStruct((PACKED_ROWS, 2 * D), jnp.float32),
    )(embT)


def _mlp_body(p_ref, w1_ref, b1_ref, w2_ref, b2_ref, o_ref):
    h = jnp.dot(p_ref[...], w1_ref[...], preferred_element_type=jnp.float32)
    h = jnp.maximum(h + b1_ref[...], 0.0)
    o = jnp.dot(h, w2_ref[...], preferred_element_type=jnp.float32)
    o = o + b2_ref[...]
    o_ref[...] = jnp.clip(o, 0.0, 100.0)


MB = 2048


def _mlp(pooled, W1, b1, W2, b2):
    return pl.pallas_call(
        _mlp_body,
        grid=(B // MB,),
        in_specs=[
            pl.BlockSpec((MB, D), lambda i: (i, 0)),
            pl.BlockSpec((D, H), lambda i: (0, 0)),
            pl.BlockSpec((1, H), lambda i: (0, 0)),
            pl.BlockSpec((H, 1), lambda i: (0, 0)),
            pl.BlockSpec((1, 1), lambda i: (0, 0)),
        ],
        out_specs=pl.BlockSpec((MB, 1), lambda i: (i, 0)),
        out_shape=jax.ShapeDtypeStruct((B, 1), jnp.float32),
    )(pooled, W1, b1.reshape(1, H), W2, b2.reshape(1, 1))


def kernel(x, embedding, W1, b1, W2, b2):
    emb_pairs = _pack(embedding.T)
    pooled = _sc_pool(x.reshape(-1), emb_pairs)
    out = _mlp(pooled, W1, b1, W2, b2)
    return out.reshape(B)


# speedup vs baseline: 1.6592x; 1.6592x over previous
import functools

import jax
import jax.numpy as jnp
from jax import lax
from jax.experimental import pallas as pl
from jax.experimental.pallas import tpu as pltpu
from jax.experimental.pallas import tpu_sc as plsc

B = 16384
S = 5
D = 64
H = 128

NC = 2
NS = 16
NW = NC * NS
BPW = B // NW
NIDX = BPW * S

CB = 16
ROWS = CB * S
NCH = BPW // CB


def _sc_pool_body(x_hbm, emb_hbm, out_hbm, idx_v, half_v, coloff_v,
                  rows_v, pooled_v, sem):
    wid = lax.axis_index("s") * NC + lax.axis_index("c")
    base = wid * BPW
    pltpu.sync_copy(x_hbm.at[pl.ds(base * S, NIDX)], idx_v)

    sh = PB.bit_length() - 1

    def prep(k, _):
        sl = pl.ds(k * 16, 16)
        v = idx_v[sl]
        half_v[sl] = lax.bitwise_or(
            lax.shift_left(lax.shift_right_logical(v, sh), sh - 1),
            lax.bitwise_and(v, PB // 2 - 1),
        )
        coloff_v[sl] = lax.shift_left(
            lax.bitwise_and(lax.shift_right_logical(v, sh - 1), 1), 6
        )
        return 0

    lax.fori_loop(0, NIDX // 16, prep, 0)

    def chunk(g, _):
        pltpu.async_copy(
            emb_hbm.at[half_v.at[pl.ds(g * ROWS, ROWS)]], rows_v, sem
        ).wait()

        def pool_one(i, _):
            r = i * S
            cv = coloff_v[pl.ds(g * ROWS + r, 16)]
            c = [cv[s] for s in range(S)]
            for q in range(D // 16):
                acc = rows_v[r, pl.ds(c[0] + q * 16, 16)]
                for s in range(1, S):
                    acc = acc + rows_v[r + s, pl.ds(c[s] + q * 16, 16)]
                pooled_v[g * CB + i, pl.ds(q * 16, 16)] = acc * (1.0 / S)
            return 0

        lax.fori_loop(0, CB, pool_one, 0)
        return 0

    lax.fori_loop(0, NCH, chunk, 0)
    pltpu.sync_copy(pooled_v, out_hbm.at[pl.ds(base, BPW)])


@functools.partial(jax.jit, static_argnames=())
def _sc_pool(x_flat, emb_pairs):
    mesh = plsc.VectorSubcoreMesh(core_axis_name="c", subcore_axis_name="s")
    return pl.kernel(
        _sc_pool_body,
        out_type=jax.ShapeDtypeStruct((B, D), jnp.float32),
        mesh=mesh,
        scratch_types=[
            pltpu.VMEM((NIDX,), jnp.int32),
            pltpu.VMEM((NIDX,), jnp.int32),
            pltpu.VMEM((NIDX + 16,), jnp.int32),
            pltpu.VMEM((ROWS, 2 * D), jnp.float32),
            pltpu.VMEM((BPW, D), jnp.float32),
            pltpu.SemaphoreType.DMA,
        ],
        compiler_params=pltpu.CompilerParams(use_tc_tiling_on_sc=True),
    )(x_flat, emb_pairs)


NP = 1000000
PB = 8192
NBLK = -(-NP // PB)
PACKED_ROWS = NBLK * (PB // 2)


def _pack_body(t_ref, o_ref):
    t = t_ref[...]
    o_ref[:, :D] = jnp.transpose(t[:, : PB // 2])
    o_ref[:, D:] = jnp.transpose(t[:, PB // 2 :])


def _pack(embT):
    return pl.pallas_call(
        _pack_body,
        grid=(NBLK,),
        in_specs=[pl.BlockSpec((D, PB), lambda i: (0, i))],
        out_specs=pl.BlockSpec((PB // 2, 2 * D), lambda i: (i, 0)),
        out_shape=jax.ShapeDtypeStruct((PACKED_ROWS, 2 * D), jnp.float32),
    )(embT)


def _mlp_body(p_ref, w1_ref, b1_ref, w2_ref, b2_ref, o_ref):
    h = jnp.dot(p_ref[...], w1_ref[...], preferred_element_type=jnp.float32)
    h = jnp.maximum(h + b1_ref[...], 0.0)
    o = jnp.dot(h, w2_ref[...], preferred_element_type=jnp.float32)
    o = o + b2_ref[...]
    o_ref[...] = jnp.clip(o, 0.0, 100.0)


MB = 2048


def _mlp(pooled, W1, b1, W2, b2):
    return pl.pallas_call(
        _mlp_body,
        grid=(B // MB,),
        in_specs=[
            pl.BlockSpec((MB, D), lambda i: (i, 0)),
            pl.BlockSpec((D, H), lambda i: (0, 0)),
            pl.BlockSpec((1, H), lambda i: (0, 0)),
            pl.BlockSpec((H, 1), lambda i: (0, 0)),
            pl.BlockSpec((1, 1), lambda i: (0, 0)),
        ],
        out_specs=pl.BlockSpec((MB, 1), lambda i: (i, 0)),
        out_shape=jax.ShapeDtypeStruct((B, 1), jnp.float32),
    )(pooled, W1, b1.reshape(1, H), W2, b2.reshape(1, 1))


def kernel(x, embedding, W1, b1, W2, b2):
    emb_pairs = _pack(embedding.T)
    pooled = _sc_pool(x.reshape(-1), emb_pairs)
    out = _mlp(pooled, W1, b1, W2, b2)
    return out.reshape(B)


# speedup vs baseline: 1.8035x; 1.0870x over previous
import functools

import jax
import jax.numpy as jnp
from jax import lax
from jax.experimental import pallas as pl
from jax.experimental.pallas import tpu as pltpu
from jax.experimental.pallas import tpu_sc as plsc

B = 16384
S = 5
D = 64
H = 128

NC = 2
NS = 16
NW = NC * NS
BPW = B // NW
NIDX = BPW * S

CB = 16
ROWS = CB * S
NCH = BPW // CB


def _sc_pool_body(xT_hbm, emb_hbm, out_hbm, idx_v, half_v, coloff_v,
                  rows_a, rows_b, pooled_v, sem_a, sem_b):
    wid = lax.axis_index("s") * NC + lax.axis_index("c")
    base = wid * BPW
    pltpu.sync_copy(xT_hbm.at[:, pl.ds(base, BPW)], idx_v)

    sh = PB.bit_length() - 1

    def prep(k, _):
        for s in range(S):
            v = idx_v[s, pl.ds(k * CB, CB)]
            half_v[pl.ds(k * ROWS + s * CB, CB)] = lax.bitwise_or(
                lax.shift_left(lax.shift_right_logical(v, sh), sh - 1),
                lax.bitwise_and(v, PB // 2 - 1),
            )
            coloff_v[pl.ds(k * ROWS + s * CB, CB)] = lax.shift_left(
                lax.bitwise_and(lax.shift_right_logical(v, sh - 1), 1), 6
            )
        return 0

    lax.fori_loop(0, NCH, prep, 0)

    def start(g, buf, sem):
        pltpu.async_copy(emb_hbm.at[half_v.at[pl.ds(g * ROWS, ROWS)]], buf, sem)

    def wait(g, buf, sem):
        pltpu.make_async_copy(
            emb_hbm.at[half_v.at[pl.ds(g * ROWS, ROWS)]], buf, sem
        ).wait()

    def pool(g, buf):
        cvs = [coloff_v[pl.ds(g * ROWS + s * CB, CB)] for s in range(S)]
        for i in range(CB):
            c = [cvs[s][i] for s in range(S)]
            for q in range(D // 16):
                acc = buf[i, pl.ds(c[0] + q * 16, 16)]
                for s in range(1, S):
                    acc = acc + buf[s * CB + i, pl.ds(c[s] + q * 16, 16)]
                pooled_v[g * CB + i, pl.ds(q * 16, 16)] = acc * (1.0 / S)

    start(0, rows_a, sem_a)

    def step(h, _):
        g = 2 * h
        wait(g, rows_a, sem_a)
        start(g + 1, rows_b, sem_b)
        pool(g, rows_a)
        wait(g + 1, rows_b, sem_b)
        start(g + 2, rows_a, sem_a)
        pool(g + 1, rows_b)
        return 0

    lax.fori_loop(0, NCH // 2 - 1, step, 0)
    g = NCH - 2
    wait(g, rows_a, sem_a)
    start(g + 1, rows_b, sem_b)
    pool(g, rows_a)
    wait(g + 1, rows_b, sem_b)
    pool(g + 1, rows_b)
    pltpu.sync_copy(pooled_v, out_hbm.at[pl.ds(base, BPW)])


@functools.partial(jax.jit, static_argnames=())
def _sc_pool(xT, emb_pairs):
    mesh = plsc.VectorSubcoreMesh(core_axis_name="c", subcore_axis_name="s")
    return pl.kernel(
        _sc_pool_body,
        out_type=jax.ShapeDtypeStruct((B, D), jnp.float32),
        mesh=mesh,
        scratch_types=[
            pltpu.VMEM((S, BPW), jnp.int32),
            pltpu.VMEM((NIDX,), jnp.int32),
            pltpu.VMEM((NIDX,), jnp.int32),
            pltpu.VMEM((ROWS, 2 * D), jnp.float32),
            pltpu.VMEM((ROWS, 2 * D), jnp.float32),
            pltpu.VMEM((BPW, D), jnp.float32),
            pltpu.SemaphoreType.DMA,
            pltpu.SemaphoreType.DMA,
        ],
        compiler_params=pltpu.CompilerParams(use_tc_tiling_on_sc=True),
    )(xT, emb_pairs)


NP = 1000000
PB = 8192
NBLK = -(-NP // PB)
PACKED_ROWS = NBLK * (PB // 2)


def _pack_body(t_ref, o_ref):
    t = t_ref[...]
    o_ref[:, :D] = jnp.transpose(t[:, : PB // 2])
    o_ref[:, D:] = jnp.transpose(t[:, PB // 2 :])


def _pack(embT):
    return pl.pallas_call(
        _pack_body,
        grid=(NBLK,),
        in_specs=[pl.BlockSpec((D, PB), lambda i: (0, i))],
        out_specs=pl.BlockSpec((PB // 2, 2 * D), lambda i: (i, 0)),
        out_shape=jax.ShapeDtypeStruct((PACKED_ROWS, 2 * D), jnp.float32),
    )(embT)


def _mlp_body(p_ref, w1_ref, b1_ref, w2_ref, b2_ref, o_ref):
    h = jnp.dot(p_ref[...], w1_ref[...], preferred_element_type=jnp.float32)
    h = jnp.maximum(h + b1_ref[...], 0.0)
    o = jnp.dot(h, w2_ref[...], preferred_element_type=jnp.float32)
    o = o + b2_ref[...]
    o_ref[...] = jnp.clip(o, 0.0, 100.0)


MB = 2048


def _mlp(pooled, W1, b1, W2, b2):
    return pl.pallas_call(
        _mlp_body,
        grid=(B // MB,),
        in_specs=[
            pl.BlockSpec((MB, D), lambda i: (i, 0)),
            pl.BlockSpec((D, H), lambda i: (0, 0)),
            pl.BlockSpec((1, H), lambda i: (0, 0)),
            pl.BlockSpec((H, 1), lambda i: (0, 0)),
            pl.BlockSpec((1, 1), lambda i: (0, 0)),
        ],
        out_specs=pl.BlockSpec((MB, 1), lambda i: (i, 0)),
        out_shape=jax.ShapeDtypeStruct((B, 1), jnp.float32),
    )(pooled, W1, b1.reshape(1, H), W2, b2.reshape(1, 1))


def kernel(x, embedding, W1, b1, W2, b2):
    emb_pairs = _pack(embedding.T)
    pooled = _sc_pool(x.T, emb_pairs)
    out = _mlp(pooled, W1, b1, W2, b2)
    return out.reshape(B)


# speedup vs baseline: 1.8258x; 1.0124x over previous
import functools

import jax
import jax.numpy as jnp
from jax import lax
from jax.experimental import pallas as pl
from jax.experimental.pallas import tpu as pltpu
from jax.experimental.pallas import tpu_sc as plsc

B = 16384
S = 5
D = 64
H = 128

NC = 2
NS = 16
NW = NC * NS
BPW = B // NW
NIDX = BPW * S

CB = 16
ROWS = CB * S
NCH = BPW // CB


def _sc_pool_body(xT_hbm, emb_hbm, out_hbm, idx_v, half_v, coloff_v,
                  rows_a, rows_b, pooled_v, sem_a, sem_b):
    wid = lax.axis_index("s") * NC + lax.axis_index("c")
    base = wid * BPW
    pltpu.sync_copy(xT_hbm.at[:, pl.ds(base, BPW)], idx_v)

    sh = PB.bit_length() - 1

    def prep(k, _):
        for s in range(S):
            v = idx_v[s, pl.ds(k * CB, CB)]
            half_v[pl.ds(k * ROWS + s * CB, CB)] = lax.bitwise_or(
                lax.shift_left(lax.shift_right_logical(v, sh), sh - 1),
                lax.bitwise_and(v, PB // 2 - 1),
            )
            coloff_v[pl.ds(k * ROWS + s * CB, CB)] = lax.shift_left(
                lax.bitwise_and(lax.shift_right_logical(v, sh - 1), 1), 6
            )
        return 0

    lax.fori_loop(0, NCH, prep, 0)

    def start(g, buf, sem):
        pltpu.async_copy(emb_hbm.at[half_v.at[pl.ds(g * ROWS, ROWS)]], buf, sem)

    def wait(g, buf, sem):
        pltpu.make_async_copy(
            emb_hbm.at[half_v.at[pl.ds(g * ROWS, ROWS)]], buf, sem
        ).wait()

    def pool(g, buf):
        cvs = [coloff_v[pl.ds(g * ROWS + s * CB, CB)] for s in range(S)]
        for i in range(CB):
            c = [cvs[s][i] for s in range(S)]
            for q in range(D // 16):
                acc = buf[i, pl.ds(c[0] + q * 16, 16)]
                for s in range(1, S):
                    acc = acc + buf[s * CB + i, pl.ds(c[s] + q * 16, 16)]
                pooled_v[g * CB + i, pl.ds(q * 16, 16)] = acc * (1.0 / S)

    start(0, rows_a, sem_a)

    def step(h, _):
        g = 2 * h
        wait(g, rows_a, sem_a)
        start(g + 1, rows_b, sem_b)
        pool(g, rows_a)
        wait(g + 1, rows_b, sem_b)
        start(g + 2, rows_a, sem_a)
        pool(g + 1, rows_b)
        return 0

    lax.fori_loop(0, NCH // 2 - 1, step, 0)
    g = NCH - 2
    wait(g, rows_a, sem_a)
    start(g + 1, rows_b, sem_b)
    pool(g, rows_a)
    wait(g + 1, rows_b, sem_b)
    pool(g + 1, rows_b)
    pltpu.sync_copy(pooled_v, out_hbm.at[pl.ds(base, BPW)])


@functools.partial(jax.jit, static_argnames=())
def _sc_pool(xT, emb_pairs):
    mesh = plsc.VectorSubcoreMesh(core_axis_name="c", subcore_axis_name="s")
    return pl.kernel(
        _sc_pool_body,
        out_type=jax.ShapeDtypeStruct((B, D), jnp.float32),
        mesh=mesh,
        scratch_types=[
            pltpu.VMEM((S, BPW), jnp.int32),
            pltpu.VMEM((NIDX,), jnp.int32),
            pltpu.VMEM((NIDX,), jnp.int32),
            pltpu.VMEM((ROWS, 2 * D), jnp.float32),
            pltpu.VMEM((ROWS, 2 * D), jnp.float32),
            pltpu.VMEM((BPW, D), jnp.float32),
            pltpu.SemaphoreType.DMA,
            pltpu.SemaphoreType.DMA,
        ],
        compiler_params=pltpu.CompilerParams(use_tc_tiling_on_sc=True),
    )(xT, emb_pairs)


NP = 1000000
PB = 8192
NBLK = -(-NP // PB)
PACKED_ROWS = NBLK * (PB // 2)


def _pack_body(t_ref, o_ref):
    t = t_ref[...]
    o_ref[:, :D] = jnp.transpose(t[:, : PB // 2])
    o_ref[:, D:] = jnp.transpose(t[:, PB // 2 :])


def _pack(embT):
    return pl.pallas_call(
        _pack_body,
        grid=(NBLK,),
        in_specs=[pl.BlockSpec((D, PB), lambda i: (0, i))],
        out_specs=pl.BlockSpec((PB // 2, 2 * D), lambda i: (i, 0)),
        out_shape=jax.ShapeDtypeStruct((PACKED_ROWS, 2 * D), jnp.float32),
    )(embT)


def _mlp_body(p_ref, w1_ref, b1_ref, w2_ref, b2_ref, o_ref):
    h = jnp.dot(p_ref[...], w1_ref[...], preferred_element_type=jnp.float32)
    h = jnp.maximum(h + b1_ref[...], 0.0)
    o = jnp.sum(h * w2_ref[...], axis=1) + b2_ref[0, 0]
    o_ref[...] = jnp.clip(o, 0.0, 100.0)


MB = 2048


def _mlp(pooled, W1, b1, W2, b2):
    return pl.pallas_call(
        _mlp_body,
        grid=(B // MB,),
        in_specs=[
            pl.BlockSpec((MB, D), lambda i: (i, 0)),
            pl.BlockSpec((D, H), lambda i: (0, 0)),
            pl.BlockSpec((1, H), lambda i: (0, 0)),
            pl.BlockSpec((1, H), lambda i: (0, 0)),
            pl.BlockSpec((1, 1), lambda i: (0, 0)),
        ],
        out_specs=pl.BlockSpec((MB,), lambda i: (i,)),
        out_shape=jax.ShapeDtypeStruct((B,), jnp.float32),
    )(pooled, W1, b1.reshape(1, H), W2.reshape(1, H), b2.reshape(1, 1))


def kernel(x, embedding, W1, b1, W2, b2):
    emb_pairs = _pack(embedding.T)
    pooled = _sc_pool(x.T, emb_pairs)
    return _mlp(pooled, W1, b1, W2, b2)


# speedup vs baseline: 1.8296x; 1.0021x over previous
import functools

import jax
import jax.numpy as jnp
from jax import lax
from jax.experimental import pallas as pl
from jax.experimental.pallas import tpu as pltpu
from jax.experimental.pallas import tpu_sc as plsc

B = 16384
S = 5
D = 64
H = 128

NC = 2
NS = 16
NW = NC * NS
BPW = B // NW
NIDX = BPW * S

CB = 16
ROWS = CB * S
NCH = BPW // CB


def _sc_pool_body(xT_hbm, emb_hbm, out_hbm, idx_v, half_v, coloff_v,
                  rows_a, rows_b, pooled_v, sem_a, sem_b):
    wid = lax.axis_index("s") * NC + lax.axis_index("c")
    base = wid * BPW
    pltpu.sync_copy(xT_hbm.at[:, pl.ds(base, BPW)], idx_v)

    sh = PB.bit_length() - 1

    def prep(k, _):
        for s in range(S):
            v = idx_v[s, pl.ds(k * CB, CB)]
            half_v[pl.ds(k * ROWS + s * CB, CB)] = lax.bitwise_or(
                lax.shift_left(lax.shift_right_logical(v, sh), sh - 1),
                lax.bitwise_and(v, PB // 2 - 1),
            )
            coloff_v[pl.ds(k * ROWS + s * CB, CB)] = lax.shift_left(
                lax.bitwise_and(lax.shift_right_logical(v, sh - 1), 1), 6
            )
        return 0

    lax.fori_loop(0, NCH, prep, 0)

    def start(g, buf, sem):
        pltpu.async_copy(emb_hbm.at[half_v.at[pl.ds(g * ROWS, ROWS)]], buf, sem)

    def wait(g, buf, sem):
        pltpu.make_async_copy(
            emb_hbm.at[half_v.at[pl.ds(g * ROWS, ROWS)]], buf, sem
        ).wait()

    def pool(g, buf):
        cvs = [coloff_v[pl.ds(g * ROWS + s * CB, CB)] for s in range(S)]
        for i in range(CB):
            c = [cvs[s][i] for s in range(S)]
            for q in range(D // 16):
                acc = buf[i, pl.ds(c[0] + q * 16, 16)]
                for s in range(1, S):
                    acc = acc + buf[s * CB + i, pl.ds(c[s] + q * 16, 16)]
                pooled_v[g * CB + i, pl.ds(q * 16, 16)] = acc * (1.0 / S)

    start(0, rows_a, sem_a)

    def step(h, _):
        g = 2 * h
        wait(g, rows_a, sem_a)
        start(g + 1, rows_b, sem_b)
        pool(g, rows_a)
        wait(g + 1, rows_b, sem_b)
        start(g + 2, rows_a, sem_a)
        pool(g + 1, rows_b)
        return 0

    lax.fori_loop(0, NCH // 2 - 1, step, 0)
    g = NCH - 2
    wait(g, rows_a, sem_a)
    start(g + 1, rows_b, sem_b)
    pool(g, rows_a)
    wait(g + 1, rows_b, sem_b)
    pool(g + 1, rows_b)
    pltpu.sync_copy(pooled_v, out_hbm.at[pl.ds(base, BPW)])


@functools.partial(jax.jit, static_argnames=())
def _sc_pool(xT, emb_pairs):
    mesh = plsc.VectorSubcoreMesh(core_axis_name="c", subcore_axis_name="s")
    return pl.kernel(
        _sc_pool_body,
        out_type=jax.ShapeDtypeStruct((B, D), jnp.float32),
        mesh=mesh,
        scratch_types=[
            pltpu.VMEM((S, BPW), jnp.int32),
            pltpu.VMEM((NIDX,), jnp.int32),
            pltpu.VMEM((NIDX,), jnp.int32),
            pltpu.VMEM((ROWS, 2 * D), jnp.float32),
            pltpu.VMEM((ROWS, 2 * D), jnp.float32),
            pltpu.VMEM((BPW, D), jnp.float32),
            pltpu.SemaphoreType.DMA,
            pltpu.SemaphoreType.DMA,
        ],
        compiler_params=pltpu.CompilerParams(use_tc_tiling_on_sc=True),
    )(xT, emb_pairs)


NP = 1000000
PB = 8192
NBLK = -(-NP // PB)
PACKED_ROWS = NBLK * (PB // 2)


def _pack_body(t_ref, o_ref):
    t = t_ref[...]
    o_ref[:, :D] = jnp.transpose(t[:, : PB // 2])
    o_ref[:, D:] = jnp.transpose(t[:, PB // 2 :])


def _pack(embT):
    return pl.pallas_call(
        _pack_body,
        grid=(NBLK,),
        in_specs=[pl.BlockSpec((D, PB), lambda i: (0, i))],
        out_specs=pl.BlockSpec((PB // 2, 2 * D), lambda i: (i, 0)),
        out_shape=jax.ShapeDtypeStruct((PACKED_ROWS, 2 * D), jnp.float32),
    )(embT)


def _mlp_body(p_ref, w1_ref, b1_ref, w2_ref, b2_ref, o_ref):
    h = jnp.dot(p_ref[...], w1_ref[...], preferred_element_type=jnp.float32)
    h = jnp.maximum(h + b1_ref[...], 0.0)
    o = jnp.dot(h, w2_ref[...], preferred_element_type=jnp.float32)[:, 0]
    o = o + b2_ref[0, 0]
    o_ref[...] = jnp.clip(o, 0.0, 100.0)


MB = 2048


def _mlp(pooled, W1, b1, W2, b2):
    return pl.pallas_call(
        _mlp_body,
        grid=(B // MB,),
        in_specs=[
            pl.BlockSpec((MB, D), lambda i: (i, 0)),
            pl.BlockSpec((D, H), lambda i: (0, 0)),
            pl.BlockSpec((1, H), lambda i: (0, 0)),
            pl.BlockSpec((H, 1), lambda i: (0, 0)),
            pl.BlockSpec((1, 1), lambda i: (0, 0)),
        ],
        out_specs=pl.BlockSpec((MB,), lambda i: (i,)),
        out_shape=jax.ShapeDtypeStruct((B,), jnp.float32),
    )(pooled, W1, b1.reshape(1, H), W2, b2.reshape(1, 1))


def kernel(x, embedding, W1, b1, W2, b2):
    emb_pairs = _pack(embedding.T)
    pooled = _sc_pool(x.T, emb_pairs)
    return _mlp(pooled, W1, b1, W2, b2)


# speedup vs baseline: 2.0295x; 1.1093x over previous
import functools

import jax
import jax.numpy as jnp
from jax import lax
from jax.experimental import pallas as pl
from jax.experimental.pallas import tpu as pltpu
from jax.experimental.pallas import tpu_sc as plsc

B = 16384
S = 5
D = 64
H = 128

NC = 2
NS = 16
NW = NC * NS
BPW = B // NW
NIDX = BPW * S

CB = 16
ROWS = CB * S
NCH = BPW // CB


def _sc_pool_body(xT_hbm, emb_hbm, out_hbm, idx_v, half_v, coloff_v,
                  rows_a, rows_b, pooled_v, sem_a, sem_b):
    wid = lax.axis_index("s") * NC + lax.axis_index("c")
    base = wid * BPW
    pltpu.sync_copy(xT_hbm.at[:, pl.ds(base, BPW)], idx_v)

    sh = PB.bit_length() - 1

    def prep(k, _):
        for s in range(S):
            v = idx_v[s, pl.ds(k * CB, CB)]
            half_v[pl.ds(k * ROWS + s * CB, CB)] = lax.bitwise_or(
                lax.shift_left(lax.shift_right_logical(v, sh), sh - 1),
                lax.bitwise_and(v, PB // 2 - 1),
            )
            coloff_v[pl.ds(k * ROWS + s * CB, CB)] = lax.shift_left(
                lax.bitwise_and(lax.shift_right_logical(v, sh - 1), 1), 6
            )
        return 0

    lax.fori_loop(0, NCH, prep, 0)

    def start(g, buf, sem):
        pltpu.async_copy(emb_hbm.at[half_v.at[pl.ds(g * ROWS, ROWS)]], buf, sem)

    def wait(g, buf, sem):
        pltpu.make_async_copy(
            emb_hbm.at[half_v.at[pl.ds(g * ROWS, ROWS)]], buf, sem
        ).wait()

    def pool(g, buf):
        cvs = [coloff_v[pl.ds(g * ROWS + s * CB, CB)] for s in range(S)]
        for i in range(CB):
            c = [cvs[s][i] for s in range(S)]
            for q in range(D // 16):
                acc = buf[i, pl.ds(c[0] + q * 16, 16)]
                for s in range(1, S):
                    acc = acc + buf[s * CB + i, pl.ds(c[s] + q * 16, 16)]
                pooled_v[g * CB + i, pl.ds(q * 16, 16)] = acc * (1.0 / S)

    start(0, rows_a, sem_a)

    def step(h, _):
        g = 2 * h
        wait(g, rows_a, sem_a)
        start(g + 1, rows_b, sem_b)
        pool(g, rows_a)
        wait(g + 1, rows_b, sem_b)
        start(g + 2, rows_a, sem_a)
        pool(g + 1, rows_b)
        return 0

    lax.fori_loop(0, NCH // 2 - 1, step, 0)
    g = NCH - 2
    wait(g, rows_a, sem_a)
    start(g + 1, rows_b, sem_b)
    pool(g, rows_a)
    wait(g + 1, rows_b, sem_b)
    pool(g + 1, rows_b)
    pltpu.sync_copy(pooled_v, out_hbm.at[pl.ds(base, BPW)])


@functools.partial(jax.jit, static_argnames=())
def _sc_pool(xT, emb_pairs):
    mesh = plsc.VectorSubcoreMesh(core_axis_name="c", subcore_axis_name="s")
    return pl.kernel(
        _sc_pool_body,
        out_type=jax.ShapeDtypeStruct((B, D), jnp.float32),
        mesh=mesh,
        scratch_types=[
            pltpu.VMEM((S, BPW), jnp.int32),
            pltpu.VMEM((NIDX,), jnp.int32),
            pltpu.VMEM((NIDX,), jnp.int32),
            pltpu.VMEM((ROWS, 2 * D), jnp.float32),
            pltpu.VMEM((ROWS, 2 * D), jnp.float32),
            pltpu.VMEM((BPW, D), jnp.float32),
            pltpu.SemaphoreType.DMA,
            pltpu.SemaphoreType.DMA,
        ],
        compiler_params=pltpu.CompilerParams(use_tc_tiling_on_sc=True),
    )(xT, emb_pairs)


NP = 1000000
PB = 16384
NBLK = -(-NP // PB)
PACKED_ROWS = NBLK * (PB // 2)


def _pack_body(t_ref, o_ref):
    t = t_ref[...]
    o_ref[:, :D] = jnp.transpose(t[:, : PB // 2])
    o_ref[:, D:] = jnp.transpose(t[:, PB // 2 :])


def _pack(embT):
    return pl.pallas_call(
        _pack_body,
        grid=(NBLK,),
        in_specs=[pl.BlockSpec((D, PB), lambda i: (0, i))],
        out_specs=pl.BlockSpec((PB // 2, 2 * D), lambda i: (i, 0)),
        out_shape=jax.ShapeDtypeStruct((PACKED_ROWS, 2 * D), jnp.float32),
    )(embT)


def _mlp_body(p_ref, w1_ref, b1_ref, w2_ref, b2_ref, o_ref):
    h = jnp.dot(p_ref[...], w1_ref[...], preferred_element_type=jnp.float32)
    h = jnp.maximum(h + b1_ref[...], 0.0)
    o = jnp.dot(h, w2_ref[...], preferred_element_type=jnp.float32)[:, 0]
    o = o + b2_ref[0, 0]
    o_ref[...] = jnp.clip(o, 0.0, 100.0)


MB = 2048


def _mlp(pooled, W1, b1, W2, b2):
    return pl.pallas_call(
        _mlp_body,
        grid=(B // MB,),
        in_specs=[
            pl.BlockSpec((MB, D), lambda i: (i, 0)),
            pl.BlockSpec((D, H), lambda i: (0, 0)),
            pl.BlockSpec((1, H), lambda i: (0, 0)),
            pl.BlockSpec((H, 1), lambda i: (0, 0)),
            pl.BlockSpec((1, 1), lambda i: (0, 0)),
        ],
        out_specs=pl.BlockSpec((MB,), lambda i: (i,)),
        out_shape=jax.ShapeDtypeStruct((B,), jnp.float32),
    )(pooled, W1, b1.reshape(1, H), W2, b2.reshape(1, 1))


def kernel(x, embedding, W1, b1, W2, b2):
    emb_pairs = _pack(embedding.T)
    pooled = _sc_pool(x.T, emb_pairs)
    return _mlp(pooled, W1, b1, W2, b2)


# speedup vs baseline: 2.1307x; 1.0498x over previous
import functools

import jax
import jax.numpy as jnp
from jax import lax
from jax.experimental import pallas as pl
from jax.experimental.pallas import tpu as pltpu
from jax.experimental.pallas import tpu_sc as plsc

B = 16384
S = 5
D = 64
H = 128

NC = 2
NS = 16
NW = NC * NS
BPW = B // NW
NIDX = BPW * S

CB = 16
ROWS = CB * S
NCH = BPW // CB


def _sc_pool_body(xT_hbm, emb_hbm, out_hbm, idx_v, half_v, coloff_v,
                  rows_a, rows_b, pooled_v, sem_a, sem_b):
    wid = lax.axis_index("s") * NC + lax.axis_index("c")
    base = wid * BPW
    pltpu.sync_copy(xT_hbm.at[:, pl.ds(base, BPW)], idx_v)

    sh = PB.bit_length() - 1

    def prep(k, _):
        for s in range(S):
            v = idx_v[s, pl.ds(k * CB, CB)]
            half_v[pl.ds(k * ROWS + s * CB, CB)] = lax.bitwise_or(
                lax.shift_left(lax.shift_right_logical(v, sh), sh - 1),
                lax.bitwise_and(v, PB // 2 - 1),
            )
            coloff_v[pl.ds(k * ROWS + s * CB, CB)] = lax.shift_left(
                lax.bitwise_and(lax.shift_right_logical(v, sh - 1), 1), 6
            )
        return 0

    lax.fori_loop(0, NCH, prep, 0)

    def start(g, buf, sem):
        pltpu.async_copy(emb_hbm.at[half_v.at[pl.ds(g * ROWS, ROWS)]], buf, sem)

    def wait(g, buf, sem):
        pltpu.make_async_copy(
            emb_hbm.at[half_v.at[pl.ds(g * ROWS, ROWS)]], buf, sem
        ).wait()

    def pool(g, buf):
        cvs = [coloff_v[pl.ds(g * ROWS + s * CB, CB)] for s in range(S)]
        for i in range(CB):
            c = [cvs[s][i] for s in range(S)]
            for q in range(D // 16):
                acc = buf[i, pl.ds(c[0] + q * 16, 16)]
                for s in range(1, S):
                    acc = acc + buf[s * CB + i, pl.ds(c[s] + q * 16, 16)]
                pooled_v[g * CB + i, pl.ds(q * 16, 16)] = acc * (1.0 / S)

    start(0, rows_a, sem_a)

    def step(h, _):
        g = 2 * h
        wait(g, rows_a, sem_a)
        start(g + 1, rows_b, sem_b)
        pool(g, rows_a)
        wait(g + 1, rows_b, sem_b)
        start(g + 2, rows_a, sem_a)
        pool(g + 1, rows_b)
        return 0

    lax.fori_loop(0, NCH // 2 - 1, step, 0)
    g = NCH - 2
    wait(g, rows_a, sem_a)
    start(g + 1, rows_b, sem_b)
    pool(g, rows_a)
    wait(g + 1, rows_b, sem_b)
    pool(g + 1, rows_b)
    pltpu.sync_copy(pooled_v, out_hbm.at[pl.ds(base, BPW)])


@functools.partial(jax.jit, static_argnames=())
def _sc_pool(xT, emb_pairs):
    mesh = plsc.VectorSubcoreMesh(core_axis_name="c", subcore_axis_name="s")
    return pl.kernel(
        _sc_pool_body,
        out_type=jax.ShapeDtypeStruct((B, D), jnp.float32),
        mesh=mesh,
        scratch_types=[
            pltpu.VMEM((S, BPW), jnp.int32),
            pltpu.VMEM((NIDX,), jnp.int32),
            pltpu.VMEM((NIDX,), jnp.int32),
            pltpu.VMEM((ROWS, 2 * D), jnp.float32),
            pltpu.VMEM((ROWS, 2 * D), jnp.float32),
            pltpu.VMEM((BPW, D), jnp.float32),
            pltpu.SemaphoreType.DMA,
            pltpu.SemaphoreType.DMA,
        ],
        compiler_params=pltpu.CompilerParams(use_tc_tiling_on_sc=True),
    )(xT, emb_pairs)


NP = 1000000
PB = 32768
NBLK = -(-NP // PB)
PACKED_ROWS = NBLK * (PB // 2)


def _pack_body(t_ref, o_ref):
    t = t_ref[...]
    o_ref[:, :D] = jnp.transpose(t[:, : PB // 2])
    o_ref[:, D:] = jnp.transpose(t[:, PB // 2 :])


def _pack(embT):
    return pl.pallas_call(
        _pack_body,
        grid=(NBLK,),
        in_specs=[pl.BlockSpec((D, PB), lambda i: (0, i))],
        out_specs=pl.BlockSpec((PB // 2, 2 * D), lambda i: (i, 0)),
        out_shape=jax.ShapeDtypeStruct((PACKED_ROWS, 2 * D), jnp.float32),
    )(embT)


def _mlp_body(p_ref, w1_ref, b1_ref, w2_ref, b2_ref, o_ref):
    h = jnp.dot(p_ref[...], w1_ref[...], preferred_element_type=jnp.float32)
    h = jnp.maximum(h + b1_ref[...], 0.0)
    o = jnp.dot(h, w2_ref[...], preferred_element_type=jnp.float32)[:, 0]
    o = o + b2_ref[0, 0]
    o_ref[...] = jnp.clip(o, 0.0, 100.0)


MB = 2048


def _mlp(pooled, W1, b1, W2, b2):
    return pl.pallas_call(
        _mlp_body,
        grid=(B // MB,),
        in_specs=[
            pl.BlockSpec((MB, D), lambda i: (i, 0)),
            pl.BlockSpec((D, H), lambda i: (0, 0)),
            pl.BlockSpec((1, H), lambda i: (0, 0)),
            pl.BlockSpec((H, 1), lambda i: (0, 0)),
            pl.BlockSpec((1, 1), lambda i: (0, 0)),
        ],
        out_specs=pl.BlockSpec((MB,), lambda i: (i,)),
        out_shape=jax.ShapeDtypeStruct((B,), jnp.float32),
    )(pooled, W1, b1.reshape(1, H), W2, b2.reshape(1, 1))


def kernel(x, embedding, W1, b1, W2, b2):
    emb_pairs = _pack(embedding.T)
    pooled = _sc_pool(x.T, emb_pairs)
    return _mlp(pooled, W1, b1, W2, b2)
